# baseline (speedup 1.0000x reference)
import functools

import jax
import jax.numpy as jnp
from jax import lax
from jax.experimental import pallas as pl
from jax.experimental.pallas import tpu as pltpu
from jax.experimental.pallas import tpu_sc as plsc

_B, _S, _D = 4, 4096, 1024
_NC, _NS = 2, 16
_NW = _NC * _NS
_SPW = _S // _NW
_CH = 16
_NCHUNK = _SPW // _CH
_TILE = _CH * _D

_mesh = plsc.VectorSubcoreMesh(core_axis_name="c", subcore_axis_name="s")


@functools.partial(
    pl.kernel,
    out_type=jax.ShapeDtypeStruct((_B * _S * _D,), jnp.float32),
    mesh=_mesh,
    scratch_types=[
        pltpu.VMEM((_TILE,), jnp.float32),
        pltpu.VMEM((_TILE,), jnp.float32),
        pltpu.VMEM((_TILE,), jnp.float32),
        pltpu.VMEM((_TILE,), jnp.float32),
        pltpu.SemaphoreType.DMA,
        pltpu.SemaphoreType.DMA,
        pltpu.SemaphoreType.DMA,
        pltpu.SemaphoreType.DMA,
        pltpu.SemaphoreType.DMA,
        pltpu.SemaphoreType.DMA,
    ],
)
def _sc_add(x_hbm, emb_hbm, out_hbm,
            x0, x1, e0, e1, si0, si1, so0, so1, se0, se1):
    wid = lax.axis_index("s") * _NC + lax.axis_index("c")
    base = wid * _SPW
    xbuf, isem, osem = (x0, x1), (si0, si1), (so0, so1)
    ebuf, esem = (e0, e1), (se0, se1)
    in_d = [None, None]
    out_d = [None, None]
    emb_d = [None, None]

    def x_off(t):
        ci, b = divmod(t, _B)
        return (b * _S + base + ci * _CH) * _D

    ntiles = _NCHUNK * _B
    emb_d[0] = pltpu.async_copy(emb_hbm.at[pl.ds(base * _D, _TILE)], e0, se0)
    in_d[0] = pltpu.async_copy(x_hbm.at[pl.ds(x_off(0), _TILE)], x0, si0)

    for t in range(ntiles):
        p = t & 1
        ci, b = divmod(t, _B)
        q = ci & 1
        if b == 0:
            if ci + 1 < _NCHUNK:
                nxt = (base + (ci + 1) * _CH) * _D
                emb_d[1 - q] = pltpu.async_copy(
                    emb_hbm.at[pl.ds(nxt, _TILE)], ebuf[1 - q], esem[1 - q])
            emb_d[q].wait()
        if t + 1 < ntiles:
            if t >= 1:
                out_d[1 - p].wait()
            in_d[1 - p] = pltpu.async_copy(
                x_hbm.at[pl.ds(x_off(t + 1), _TILE)], xbuf[1 - p], isem[1 - p])
        in_d[p].wait()

        xb, eb = xbuf[p], ebuf[q]

        @plsc.parallel_loop(0, _TILE, step=16, unroll=8)
        def _add(i):
            xb[pl.ds(i, 16)] = xb[pl.ds(i, 16)] + eb[pl.ds(i, 16)]

        out_d[p] = pltpu.async_copy(
            xbuf[p], out_hbm.at[pl.ds(x_off(t), _TILE)], osem[p])

    out_d[0].wait()
    out_d[1].wait()


@jax.jit
def kernel(x, emb):
    B, S, D = x.shape
    out = _sc_add(x.reshape(-1), emb.reshape(-1))
    return out.reshape(B, S, D)
